# baseline (speedup 1.0000x reference)
import functools

import jax
import jax.numpy as jnp
from jax import lax
from jax.experimental import pallas as pl
from jax.experimental.pallas import tpu as pltpu
from jax.experimental.pallas import tpu_sc as plsc

B = 16384
D = 32
NC, NS = 2, 16
NW = NC * NS
BPW = B // NW
CH = 128
NCHUNK = BPW // CH

USER_V, ORG_V = 1000000, 100000
FIELD_V, ROLE_V, DAY_V, HOUR_V = 1000, 1000, 32, 24

_mesh = plsc.VectorSubcoreMesh(
    core_axis_name="c", subcore_axis_name="s", num_cores=NC, num_subcores=NS
)


def _tc_gather_body(uid_ref, oid_ref, ublk, oblk, uout, oout):
    lane = jax.lax.broadcasted_iota(jnp.int32, (32, 128), 1)
    i = pl.program_id(0)
    row = i % 8
    ucol = uid_ref[i] % 128
    uval = jnp.sum(jnp.where(lane == ucol, ublk[...], 0.0), axis=1)
    uout[pl.ds(row, 1), :] = uval[None, :]
    ocol = oid_ref[i] % 128
    oval = jnp.sum(jnp.where(lane == ocol, oblk[...], 0.0), axis=1)
    oout[pl.ds(row, 1), :] = oval[None, :]


@jax.jit
def _tc_gather(WuT, WoT, uid, oid):
    grid_spec = pltpu.PrefetchScalarGridSpec(
        num_scalar_prefetch=2,
        grid=(B,),
        in_specs=[
            pl.BlockSpec((32, 128),
                         lambda i, uids, oids: (0, uids[i] // 128)),
            pl.BlockSpec((32, 128),
                         lambda i, uids, oids: (0, oids[i] // 128)),
        ],
        out_specs=[
            pl.BlockSpec((8, 32), lambda i, uids, oids: (i // 8, 0)),
            pl.BlockSpec((8, 32), lambda i, uids, oids: (i // 8, 0)),
        ],
    )
    return pl.pallas_call(
        _tc_gather_body,
        grid_spec=grid_spec,
        out_shape=[
            jax.ShapeDtypeStruct((B, D), jnp.float32),
            jax.ShapeDtypeStruct((B, D), jnp.float32),
        ],
    )(uid, oid, WuT, WoT)


@functools.partial(
    pl.kernel,
    out_type=jax.ShapeDtypeStruct((B, 5 * D), jnp.float32),
    mesh=_mesh,
    scratch_types=[
        pltpu.VMEM((5, BPW), jnp.int32),
        pltpu.VMEM((FIELD_V * D // 128, 128), jnp.float32),
        pltpu.VMEM((ROLE_V * D // 128, 128), jnp.float32),
        pltpu.VMEM((DAY_V * D // 128, 128), jnp.float32),
        pltpu.VMEM((HOUR_V * D // 128, 128), jnp.float32),
        pltpu.VMEM((2, CH, D), jnp.float32),
        pltpu.SemaphoreType.DMA,
        pltpu.SemaphoreType.DMA,
        pltpu.SemaphoreType.DMA,
    ],
    compiler_params=pltpu.CompilerParams(use_tc_tiling_on_sc=False),
)
def _sc_smalls(f0, f1, r, d, t, wf, wr, wd, wh,
               out, idx_v, wf_v, wr_v, wd_v, wh_v, ext_v,
               sem_i, sem_t, sem_o):
    wid = lax.axis_index("s") * NC + lax.axis_index("c")
    base = wid * BPW
    idx_hbm = (f0, f1, r, d, t)
    icps = [
        pltpu.async_copy(idx_hbm[i].at[pl.ds(base, BPW)], idx_v.at[i], sem_i)
        for i in range(5)
    ]
    tcps = [
        pltpu.async_copy(src, dst, sem_t)
        for src, dst in ((wf, wf_v), (wr, wr_v), (wd, wd_v), (wh, wh_v))
    ]
    for c in icps:
        c.wait()
    for c in tcps:
        c.wait()

    smalls = ((0, wf_v), (1, wf_v), (2, wr_v), (3, wd_v), (4, wh_v))
    ocp = [None, None]
    for si, (feat, tab) in enumerate(smalls):
        for h in range(NCHUNK):
            slot = (si * NCHUNK + h) % 2
            if ocp[slot] is not None:
                ocp[slot].wait()

            def sbody(g, carry, feat=feat, tab=tab, h=h, slot=slot):
                idx16 = idx_v[feat, pl.ds(h * CH + g * 16, 16)] * D
                for k in range(16):
                    s = idx16[k]
                    q = lax.shift_right_logical(s, 7)
                    cc = s & 127
                    rr = g * 16 + k
                    ext_v[slot, rr, pl.ds(0, 16)] = tab[q, pl.ds(cc, 16)]
                    ext_v[slot, rr, pl.ds(16, 16)] = tab[q,
                                                         pl.ds(cc + 16, 16)]
                return carry

            lax.fori_loop(0, CH // 16, sbody, 0)
            ocp[slot] = pltpu.async_copy(
                ext_v.at[slot],
                out.at[pl.ds(base + h * CH, CH),
                       pl.ds(feat * D, D)], sem_o)
    for slot in (0, 1):
        if ocp[slot] is not None:
            ocp[slot].wait()


def kernel(user_id, organization, interested_fields_0, interested_fields_1,
           role, date, time, W_user, W_org, W_field, W_role, W_day, W_hour):
    e_user, e_org = _tc_gather(W_user.T, W_org.T,
                               user_id.astype(jnp.int32),
                               organization.astype(jnp.int32))
    sc_out = _sc_smalls(
        interested_fields_0, interested_fields_1, role, date, time,
        W_field.reshape(FIELD_V * D // 128, 128),
        W_role.reshape(ROLE_V * D // 128, 128),
        W_day.reshape(DAY_V * D // 128, 128),
        W_hour.reshape(HOUR_V * D // 128, 128))
    return jnp.concatenate([e_user, e_org, sc_out], axis=1)


# speedup vs baseline: 12.9971x; 12.9971x over previous
import functools

import jax
import jax.numpy as jnp
from jax import lax
from jax.experimental import pallas as pl
from jax.experimental.pallas import tpu as pltpu
from jax.experimental.pallas import tpu_sc as plsc

B = 16384
D = 32
NC, NS = 2, 16
NW = NC * NS
BPW = B // NW
HCH = BPW // 2

FIELD_V, ROLE_V, DAY_V, HOUR_V = 1000, 1000, 32, 24

_mesh = plsc.VectorSubcoreMesh(
    core_axis_name="c", subcore_axis_name="s", num_cores=NC, num_subcores=NS
)


@functools.partial(
    pl.kernel,
    out_type=jax.ShapeDtypeStruct((B, 7 * D), jnp.float32),
    mesh=_mesh,
    scratch_types=[
        pltpu.VMEM((7, BPW), jnp.int32),
        pltpu.VMEM((BPW, D), jnp.float32),
        pltpu.VMEM((BPW, D), jnp.float32),
        pltpu.VMEM((FIELD_V * D // 128, 128), jnp.float32),
        pltpu.VMEM((ROLE_V * D // 128, 128), jnp.float32),
        pltpu.VMEM((DAY_V * D // 128, 128), jnp.float32),
        pltpu.VMEM((HOUR_V * D // 128, 128), jnp.float32),
        pltpu.VMEM((HCH, D), jnp.float32),
        pltpu.VMEM((HCH, D), jnp.float32),
        pltpu.SemaphoreType.DMA,
        pltpu.SemaphoreType.DMA,
        pltpu.SemaphoreType.DMA,
        pltpu.SemaphoreType.DMA,
    ],
    compiler_params=pltpu.CompilerParams(use_tc_tiling_on_sc=False),
)
def _usermodel(u, o, f0, f1, r, d, t, Wu, Wo, wf, wr, wd, wh,
               out, idx_v, rows_u, rows_o, wf_v, wr_v, wd_v, wh_v,
               ext_a, ext_b, sem_i, sem_g, sem_t, sem_o):
    wid = lax.axis_index("s") * NC + lax.axis_index("c")
    base = wid * BPW
    idx_hbm = (u, o, f0, f1, r, d, t)

    icps = [
        pltpu.async_copy(idx_hbm[i].at[pl.ds(base, BPW)], idx_v.at[i], sem_i)
        for i in range(7)
    ]
    tcps = [
        pltpu.async_copy(src, dst, sem_t)
        for src, dst in ((wf, wf_v), (wr, wr_v), (wd, wd_v), (wh, wh_v))
    ]
    for c in icps:
        c.wait()
    gu = pltpu.async_copy(Wu.at[idx_v.at[0]], rows_u, sem_g)
    go = pltpu.async_copy(Wo.at[idx_v.at[1]], rows_o, sem_g)
    for c in tcps:
        c.wait()

    smalls = ((2, wf_v), (3, wf_v), (4, wr_v), (5, wd_v), (6, wh_v))
    prev = [None, None]
    nslot = 0
    for si, (feat, tab) in enumerate(smalls):
        for half in range(2):
            slot = nslot % 2
            nslot += 1
            buf = ext_a if slot == 0 else ext_b
            if prev[slot] is not None:
                prev[slot].wait()

            def body(g, carry, feat=feat, tab=tab, half=half, buf=buf):
                idx16 = idx_v[feat, pl.ds(half * HCH + g * 16, 16)] * D
                for k in range(16):
                    s = idx16[k]
                    q = lax.shift_right_logical(s, 7)
                    cc = s & 127
                    rr = g * 16 + k
                    buf[rr, pl.ds(0, 16)] = tab[q, pl.ds(cc, 16)]
                    buf[rr, pl.ds(16, 16)] = tab[q, pl.ds(cc + 16, 16)]
                return carry

            lax.fori_loop(0, HCH // 16, body, 0)
            prev[slot] = pltpu.async_copy(
                buf,
                out.at[pl.ds(base + half * HCH, HCH),
                       pl.ds(feat * D, D)], sem_o)

    gu.wait()
    cu = pltpu.async_copy(rows_u, out.at[pl.ds(base, BPW), pl.ds(0, D)],
                          sem_o)
    go.wait()
    co = pltpu.async_copy(rows_o, out.at[pl.ds(base, BPW), pl.ds(D, D)],
                          sem_o)
    for c in (prev[0], prev[1], cu, co):
        if c is not None:
            c.wait()


def kernel(user_id, organization, interested_fields_0, interested_fields_1,
           role, date, time, W_user, W_org, W_field, W_role, W_day, W_hour):
    return _usermodel(
        user_id, organization, interested_fields_0, interested_fields_1,
        role, date, time, W_user, W_org,
        W_field.reshape(FIELD_V * D // 128, 128),
        W_role.reshape(ROLE_V * D // 128, 128),
        W_day.reshape(DAY_V * D // 128, 128),
        W_hour.reshape(HOUR_V * D // 128, 128))
